# baseline (speedup 1.0000x reference)
import functools
import math

import jax
import jax.numpy as jnp
from jax import lax
from jax.experimental import pallas as pl
from jax.experimental.pallas import tpu as pltpu
from jax.experimental.pallas import tpu_sc as plsc

PAD = 0
SMOOTH = 0.1
CONF = 1.0 - SMOOTH

ROWS_BLK = 128

_INFO = plsc.get_sparse_core_info()
_NC, _NS, _L = _INFO.num_cores, _INFO.num_subcores, _INFO.num_lanes
_NW = _NC * _NS


def _tc_block(pred_ref, tgt_ref, sc_ref, out_ref):
    i = pl.program_id(0)

    x = pred_ref[...]
    t = tgt_ref[0, 0, :]
    v = x.shape[1]
    eps = SMOOTH / (v - 2)
    k_const = CONF * math.log(CONF) + SMOOTH * math.log(eps)

    row_sum = jnp.sum(x, axis=1)
    p_0 = x[:, 0]

    valid = (t != PAD)
    per_row = k_const + eps * p_0 - eps * row_sum
    partial = jnp.sum(jnp.where(valid, per_row, 0.0))

    @pl.when(i == 0)
    def _init():
        out_ref[...] = ((eps - CONF) * jnp.sum(sc_ref[...])).reshape(1, 1)

    out_ref[...] += partial.reshape(1, 1)


_TPR = 250


def _sc_gather(pred_flat_hbm, tgt_hbm, out_hbm, tgt_v, idx_v, gat_v, acc_v, sem):
    n = tgt_hbm.shape[0]
    chunk = n // _NW
    nq = chunk // _L
    wid = lax.axis_index("s") * _NC + lax.axis_index("c")
    base = wid * chunk
    pltpu.sync_copy(tgt_hbm.at[pl.ds(base, chunk)], tgt_v)
    lane = lax.iota(jnp.int32, _L)

    for q in range(nq):
        tv = tgt_v[pl.ds(q * _L, _L)]
        iv = base + q * _L + lane
        phys = (((iv >> 3) * _TPR + (tv >> 7)) << 10) \
            + ((iv & 7) << 7) + (tv & 127)
        idx_v[pl.ds(q * _L, _L)] = phys

    for h in range(chunk // 128):
        pltpu.async_copy(
            pred_flat_hbm.at[idx_v.at[pl.ds(h * 128, 128)]],
            gat_v.at[pl.ds(h * 128, 128)], sem).wait()

    acc = jnp.zeros((_L,), jnp.float32)
    for q in range(nq):
        tv = tgt_v[pl.ds(q * _L, _L)]
        gv = gat_v[pl.ds(q * _L, _L)]
        acc = acc + jnp.where(tv == PAD, 0.0, gv)
    acc_v[...] = acc
    pltpu.sync_copy(acc_v, out_hbm.at[pl.ds(wid * _L, _L)])


@jax.jit
def kernel(predicts, target):
    n, v = predicts.shape
    eps = SMOOTH / (v - 2)
    grid = n // ROWS_BLK
    tgt = target.astype(jnp.int32)
    tgt3 = tgt.reshape(grid, 1, ROWS_BLK)

    pred_flat = predicts.reshape(n // 8, 8, v // 128, 128) \
        .transpose(0, 2, 1, 3).reshape(-1)

    sc_kernel = functools.partial(
        pl.kernel,
        mesh=plsc.VectorSubcoreMesh(core_axis_name="c", subcore_axis_name="s"),
        out_type=jax.ShapeDtypeStruct((_NW * _L,), jnp.float32),
        scratch_types=[
            pltpu.VMEM((n // _NW,), jnp.int32),
            pltpu.VMEM((n // _NW,), jnp.int32),
            pltpu.VMEM((n // _NW,), jnp.float32),
            pltpu.VMEM((_L,), jnp.float32),
            pltpu.SemaphoreType.DMA,
        ],
    )(_sc_gather)
    sc_part = sc_kernel(pred_flat, tgt)

    tc_out = pl.pallas_call(
        _tc_block,
        grid=(grid,),
        in_specs=[
            pl.BlockSpec((ROWS_BLK, v), lambda i: (i, 0)),
            pl.BlockSpec((1, 1, ROWS_BLK), lambda i: (i, 0, 0)),
            pl.BlockSpec((_NW * _L,), lambda i: (0,)),
        ],
        out_specs=pl.BlockSpec((1, 1), lambda i: (0, 0)),
        out_shape=jax.ShapeDtypeStruct((1, 1), jnp.float32),
    )(predicts, tgt3, sc_part)

    return tc_out[0, 0]


# speedup vs baseline: 1.0090x; 1.0090x over previous
import functools
import math

import jax
import jax.numpy as jnp
from jax import lax
from jax.experimental import pallas as pl
from jax.experimental.pallas import tpu as pltpu
from jax.experimental.pallas import tpu_sc as plsc

PAD = 0
SMOOTH = 0.1
CONF = 1.0 - SMOOTH

ROWS_BLK = 128

_INFO = plsc.get_sparse_core_info()
_NC, _NS, _L = _INFO.num_cores, _INFO.num_subcores, _INFO.num_lanes
_NW = _NC * _NS


def _tc_block(pred_ref, tgt_ref, out_ref):
    i = pl.program_id(0)

    x = pred_ref[...]
    t = tgt_ref[0, 0, :]
    v = x.shape[1]
    eps = SMOOTH / (v - 2)
    k_const = CONF * math.log(CONF) + SMOOTH * math.log(eps)

    row_sum = jnp.sum(x, axis=1)
    p_0 = x[:, 0]

    valid = (t != PAD)
    per_row = k_const + eps * p_0 - eps * row_sum
    partial = jnp.sum(jnp.where(valid, per_row, 0.0))

    @pl.when(i == 0)
    def _init():
        out_ref[...] = jnp.zeros((1, 1), jnp.float32)

    out_ref[...] += partial.reshape(1, 1)


_TPR = 250


def _sc_gather(pred_flat_hbm, tgt_hbm, out_hbm, tgt_v, idx_v, gat_v, acc_v, sem):
    n = tgt_hbm.shape[0]
    chunk = n // _NW
    nq = chunk // _L
    wid = lax.axis_index("s") * _NC + lax.axis_index("c")
    base = wid * chunk
    pltpu.sync_copy(tgt_hbm.at[pl.ds(base, chunk)], tgt_v)
    lane = lax.iota(jnp.int32, _L)

    for q in range(nq):
        tv = tgt_v[pl.ds(q * _L, _L)]
        iv = base + q * _L + lane
        phys = (((iv >> 3) * _TPR + (tv >> 7)) << 10) \
            + ((iv & 7) << 7) + (tv & 127)
        idx_v[pl.ds(q * _L, _L)] = phys

    for h in range(chunk // 128):
        pltpu.async_copy(
            pred_flat_hbm.at[idx_v.at[pl.ds(h * 128, 128)]],
            gat_v.at[pl.ds(h * 128, 128)], sem).wait()

    acc = jnp.zeros((_L,), jnp.float32)
    for q in range(nq):
        tv = tgt_v[pl.ds(q * _L, _L)]
        gv = gat_v[pl.ds(q * _L, _L)]
        acc = acc + jnp.where(tv == PAD, 0.0, gv)
    acc_v[...] = acc
    pltpu.sync_copy(acc_v, out_hbm.at[pl.ds(wid * _L, _L)])


@jax.jit
def kernel(predicts, target):
    n, v = predicts.shape
    eps = SMOOTH / (v - 2)
    grid = n // ROWS_BLK
    tgt = target.astype(jnp.int32)
    tgt3 = tgt.reshape(grid, 1, ROWS_BLK)

    pred_flat = predicts.reshape(n // 8, 8, v // 128, 128) \
        .transpose(0, 2, 1, 3).reshape(-1)

    sc_kernel = functools.partial(
        pl.kernel,
        mesh=plsc.VectorSubcoreMesh(core_axis_name="c", subcore_axis_name="s"),
        out_type=jax.ShapeDtypeStruct((_NW * _L,), jnp.float32),
        scratch_types=[
            pltpu.VMEM((n // _NW,), jnp.int32),
            pltpu.VMEM((n // _NW,), jnp.int32),
            pltpu.VMEM((n // _NW,), jnp.float32),
            pltpu.VMEM((_L,), jnp.float32),
            pltpu.SemaphoreType.DMA,
        ],
    )(_sc_gather)
    sc_part = sc_kernel(pred_flat, tgt)

    tc_out = pl.pallas_call(
        _tc_block,
        grid=(grid,),
        in_specs=[
            pl.BlockSpec((ROWS_BLK, v), lambda i: (i, 0)),
            pl.BlockSpec((1, 1, ROWS_BLK), lambda i: (i, 0, 0)),
        ],
        out_specs=pl.BlockSpec((1, 1), lambda i: (0, 0)),
        out_shape=jax.ShapeDtypeStruct((1, 1), jnp.float32),
    )(predicts, tgt3)

    return tc_out[0, 0] + (eps - CONF) * jnp.sum(sc_part)
